# baseline (speedup 1.0000x reference)
import functools

import jax
import jax.numpy as jnp
from jax import lax
from jax.experimental import pallas as pl
from jax.experimental.pallas import tpu as pltpu
from jax.experimental.pallas import tpu_sc as plsc

NUM_NUMERIC = 16
NUM_STYLES = 26
EMB = 64
BATCH = 16384
NFEAT = NUM_NUMERIC + NUM_STYLES

NC = 2
NS = 16
NW = NC * NS
B_PER_W = BATCH // NW
CH = 128
N_CH = B_PER_W // CH

def _sc_gather_body(uid_hbm, pid_hbm, ut_hbm, pt_hbm, uout_hbm, pout_hbm,
                    idx_u, idx_p, rows_u, rows_p, sem_u, sem_p):
    wid = lax.axis_index("s") * NC + lax.axis_index("c")
    base = wid * B_PER_W
    pltpu.sync_copy(uid_hbm.at[wid], idx_u)
    pltpu.sync_copy(pid_hbm.at[wid], idx_p)
    copies = []
    for j in range(N_CH):
        copies.append(pltpu.async_copy(ut_hbm.at[idx_u.at[j]], rows_u.at[j], sem_u))
        copies.append(pltpu.async_copy(pt_hbm.at[idx_p.at[j]], rows_p.at[j], sem_p))
    for c in copies:
        c.wait()
    for j in range(N_CH):
        pltpu.sync_copy(rows_u.at[j], uout_hbm.at[pl.ds(base + j * CH, CH)])
        pltpu.sync_copy(rows_p.at[j], pout_hbm.at[pl.ds(base + j * CH, CH)])


@functools.cache
def _sc_gather():
    mesh = plsc.VectorSubcoreMesh(core_axis_name="c", subcore_axis_name="s")
    return pl.kernel(
        _sc_gather_body,
        out_type=[
            jax.ShapeDtypeStruct((BATCH, EMB), jnp.float32),
            jax.ShapeDtypeStruct((BATCH, EMB), jnp.float32),
        ],
        mesh=mesh,
        scratch_types=[
            pltpu.VMEM((N_CH, CH), jnp.int32),
            pltpu.VMEM((N_CH, CH), jnp.int32),
            pltpu.VMEM((N_CH, CH, EMB), jnp.float32),
            pltpu.VMEM((N_CH, CH, EMB), jnp.float32),
            pltpu.SemaphoreType.DMA,
            pltpu.SemaphoreType.DMA,
        ],
        compiler_params=pltpu.CompilerParams(use_tc_tiling_on_sc=False),
    )


TB = 2048


def _mlp_body(uv_ref, pv_ref, ff_ref, wfeat_ref, bns_ref, w1_ref, b1_ref,
              w2_ref, b2_ref, w3_ref, b3_ref, wf_ref, bf_ref, out_ref):
    ns = jnp.maximum(ff_ref[...] @ wfeat_ref[...] + bns_ref[...], 0.0)
    h = uv_ref[...] @ w1_ref[0:EMB, :]
    h = h + pv_ref[...] @ w1_ref[EMB:2 * EMB, :]
    h = h + ns @ w1_ref[2 * EMB:4 * EMB, :]
    h1 = jnp.maximum(h + b1_ref[...], 0.0)
    h2 = jnp.maximum(h1 @ w2_ref[...] + b2_ref[...], 0.0)
    h3 = jnp.maximum(h2 @ w3_ref[...] + b3_ref[...], 0.0)
    z = jnp.sum(h3 * wf_ref[...], axis=1, keepdims=True) + bf_ref[0, 0]
    out_ref[...] = 1.0 / (1.0 + jnp.exp(-z))


def _mlp(uvec, pvec, ff, wfeat, bns, W1, b1, W2, b2, W3, b3, wf, bf):
    grid = (BATCH // TB,)
    full = lambda shape: pl.BlockSpec(shape, lambda i: (0, 0))
    return pl.pallas_call(
        _mlp_body,
        grid=grid,
        in_specs=[
            pl.BlockSpec((TB, EMB), lambda i: (i, 0)),
            pl.BlockSpec((TB, EMB), lambda i: (i, 0)),
            pl.BlockSpec((TB, NFEAT), lambda i: (i, 0)),
            full((NFEAT, 2 * EMB)),
            full((1, 2 * EMB)),
            full((4 * EMB, 128)),
            full((1, 128)),
            full((128, 64)),
            full((1, 64)),
            full((64, 32)),
            full((1, 32)),
            full((1, 32)),
            full((1, 1)),
        ],
        out_specs=pl.BlockSpec((TB, 1), lambda i: (i, 0)),
        out_shape=jax.ShapeDtypeStruct((BATCH, 1), jnp.float32),
        compiler_params=pltpu.CompilerParams(
            dimension_semantics=("parallel",),
        ),
    )(uvec, pvec, ff, wfeat, bns, W1, b1, W2, b2, W3, b3, wf, bf)


def kernel(user_id, product_id, full_features, user_table, product_table,
           W_num, b_num, W_style, b_style, W1, b1, W2, b2, W3, b3, Wf, bf):
    uid3 = user_id.astype(jnp.int32).reshape(NW, N_CH, CH)
    pid3 = product_id.astype(jnp.int32).reshape(NW, N_CH, CH)
    uvec, pvec = _sc_gather()(uid3, pid3, user_table, product_table)

    wfeat = jnp.zeros((NFEAT, 2 * EMB), jnp.float32)
    wfeat = wfeat.at[:NUM_NUMERIC, :EMB].set(W_num)
    wfeat = wfeat.at[NUM_NUMERIC:, EMB:].set(W_style)
    bns = jnp.concatenate([b_num, b_style])[None, :]

    return _mlp(uvec, pvec, full_features, wfeat, bns,
                W1, b1[None, :], W2, b2[None, :], W3, b3[None, :],
                Wf.reshape(1, 32), bf.reshape(1, 1))
